# Initial kernel scaffold; baseline (speedup 1.0000x reference)
import jax
import jax.numpy as jnp
from jax.experimental import pallas as pl


def kernel(x, edge_index, edge_attr, W_fc, W_edge):
    raise NotImplementedError("write your pallas kernel here")


# speedup vs baseline: 3.9175x; 3.9175x over previous
import functools

import jax
import jax.numpy as jnp
from jax import lax
from jax.experimental import pallas as pl
from jax.experimental.pallas import tpu as pltpu
from jax.experimental.pallas import tpu_sc as plsc

N_NODES = 10000
N_EDGES = 320000
D = 128
DE = 4
K = 128
NC = 2
NS = 16
SLC = 640
SLC_LAST = N_NODES - 15 * SLC
CHUNKS = N_EDGES // K
CHUNKS_PER_CORE = CHUNKS // NC
TRIPS = (CHUNKS_PER_CORE + NS - 1) // NS


def _sc_scatter(x, row, col, ed2, z128):
    mesh = plsc.VectorSubcoreMesh(core_axis_name="c", subcore_axis_name="s")

    @functools.partial(
        pl.kernel,
        out_type=jax.ShapeDtypeStruct((NC * N_NODES, D), jnp.float32),
        mesh=mesh,
        scratch_types=[
            pltpu.VMEM_SHARED((N_NODES, D), jnp.float32),
            pltpu.VMEM((K,), jnp.int32),
            pltpu.VMEM((K,), jnp.int32),
            pltpu.VMEM((K, D), jnp.float32),
            pltpu.VMEM((K, D), jnp.float32),
            pltpu.SemaphoreType.DMA,
        ],
    )
    def k(x_hbm, row_hbm, col_hbm, ed2_hbm, z128_hbm,
          acc_out,
          acc_s, idx_r, idx_c, xb, edb, sem):
        c = lax.axis_index("c")
        s = lax.axis_index("s")
        r0 = s * SLC

        @pl.when(s < NS - 1)
        def _():
            pltpu.sync_copy(z128_hbm.at[pl.ds(r0, SLC)],
                            acc_s.at[pl.ds(r0, SLC)])

        @pl.when(s == NS - 1)
        def _():
            pltpu.sync_copy(z128_hbm.at[pl.ds(r0, SLC_LAST)],
                            acc_s.at[pl.ds(r0, SLC_LAST)])

        plsc.subcore_barrier()

        def body(j, carry):
            lc = j * NS + s

            @pl.when(lc < CHUNKS_PER_CORE)
            def _():
                base = (c * CHUNKS_PER_CORE + lc) * K
                pltpu.sync_copy(row_hbm.at[pl.ds(base, K)], idx_r)
                pltpu.sync_copy(col_hbm.at[pl.ds(base, K)], idx_c)
                pltpu.sync_copy(ed2_hbm.at[pl.ds(base, K)], edb)
                pltpu.async_copy(x_hbm.at[idx_c], xb, sem).wait()
                pltpu.sync_copy(xb, acc_s.at[idx_r], add=True)
                pltpu.async_copy(x_hbm.at[idx_r], xb, sem).wait()
                pltpu.sync_copy(xb, acc_s.at[idx_c], add=True)
                pltpu.sync_copy(edb, acc_s.at[idx_r], add=True)
                pltpu.sync_copy(edb, acc_s.at[idx_c], add=True)

            return carry

        lax.fori_loop(0, TRIPS, body, 0)
        plsc.subcore_barrier()
        o0 = c * N_NODES + r0

        @pl.when(s < NS - 1)
        def _():
            pltpu.sync_copy(acc_s.at[pl.ds(r0, SLC)],
                            acc_out.at[pl.ds(o0, SLC)])

        @pl.when(s == NS - 1)
        def _():
            pltpu.sync_copy(acc_s.at[pl.ds(r0, SLC_LAST)],
                            acc_out.at[pl.ds(o0, SLC_LAST)])

    return k(x, row, col, ed2, z128)


EBLK = 4000


def _tc_edge_proj(edge_attr, we2):
    def body(ea_ref, w_ref, o_ref):
        o_ref[...] = lax.dot_general(
            ea_ref[...], w_ref[...], (((1,), (1,)), ((), ())),
            preferred_element_type=jnp.float32)

    return pl.pallas_call(
        body,
        grid=(N_EDGES // EBLK,),
        in_specs=[
            pl.BlockSpec((EBLK, DE), lambda i: (i, 0)),
            pl.BlockSpec((D, DE), lambda i: (0, 0)),
        ],
        out_specs=pl.BlockSpec((EBLK, D), lambda i: (i, 0)),
        out_shape=jax.ShapeDtypeStruct((N_EDGES, D), jnp.float32),
    )(edge_attr, we2)


BLK = 1000


def _tc_combine(acc2, wfc):
    def body(a_ref, wfc_ref, o_ref):
        u = 2.0 * (a_ref[0] + a_ref[1])
        o_ref[...] = lax.dot_general(
            u, wfc_ref[...], (((1,), (1,)), ((), ())),
            preferred_element_type=jnp.float32)

    return pl.pallas_call(
        body,
        grid=(N_NODES // BLK,),
        in_specs=[
            pl.BlockSpec((NC, BLK, D), lambda i: (0, i, 0)),
            pl.BlockSpec((D, D), lambda i: (0, 0)),
        ],
        out_specs=pl.BlockSpec((BLK, D), lambda i: (i, 0)),
        out_shape=jax.ShapeDtypeStruct((N_NODES, D), jnp.float32),
    )(acc2, wfc)


def kernel(x, edge_index, edge_attr, W_fc, W_edge):
    row = edge_index[0].astype(jnp.int32)
    col = edge_index[1].astype(jnp.int32)
    z128 = jnp.zeros((N_NODES, D), jnp.float32)
    ed2 = _tc_edge_proj(edge_attr, 2.0 * W_edge)
    acc_flat = _sc_scatter(x, row, col, ed2, z128)
    acc2 = acc_flat.reshape(NC, N_NODES, D)
    return _tc_combine(acc2, W_fc)


# speedup vs baseline: 6.7433x; 1.7213x over previous
import functools

import jax
import jax.numpy as jnp
from jax import lax
from jax.experimental import pallas as pl
from jax.experimental.pallas import tpu as pltpu
from jax.experimental.pallas import tpu_sc as plsc

N_NODES = 10000
N_PAD = 16
N_TOT = N_NODES + N_PAD
N_EDGES = 320000
E_PAD = 327680
D = 128
DE = 4
K = 128
NC = 2
NS = 16
GROUPS = E_PAD // (8 * K)
GROUPS_PER_CORE = GROUPS // NC
GROUPS_PER_TILE = GROUPS_PER_CORE // NS
SLC = 640
SLC_LAST_Z = N_TOT - 15 * SLC
SLC_LAST_W = N_NODES - 15 * SLC
TW = 40960


def _sc_aggregate(x, row2d, col2d, z128):
    mesh = plsc.VectorSubcoreMesh(core_axis_name="c", subcore_axis_name="s")

    @functools.partial(
        pl.kernel,
        out_type=jax.ShapeDtypeStruct((NC * N_NODES, D), jnp.float32),
        mesh=mesh,
        scratch_types=[
            pltpu.VMEM_SHARED((N_TOT, D), jnp.float32),
            pltpu.VMEM((8, K), jnp.int32),
            pltpu.VMEM((8, K), jnp.int32),
            pltpu.VMEM((K, D), jnp.float32),
            pltpu.VMEM((K, D), jnp.float32),
            pltpu.SemaphoreType.DMA,
            pltpu.SemaphoreType.DMA,
            pltpu.SemaphoreType.DMA,
            pltpu.SemaphoreType.DMA,
        ],
    )
    def k(x_hbm, row_hbm, col_hbm, z128_hbm,
          acc_out,
          acc_s, idxr, idxc, xb0, xb1, sg0, sg1, ss0, ss1):
        c = lax.axis_index("c")
        s = lax.axis_index("s")
        r0 = s * SLC

        @pl.when(s < NS - 1)
        def _():
            pltpu.sync_copy(z128_hbm.at[pl.ds(r0, SLC)],
                            acc_s.at[pl.ds(r0, SLC)])

        @pl.when(s == NS - 1)
        def _():
            pltpu.sync_copy(z128_hbm.at[pl.ds(r0, SLC_LAST_Z)],
                            acc_s.at[pl.ds(r0, SLC_LAST_Z)])

        plsc.subcore_barrier()

        bufs = (xb0, xb1)
        gsem = (sg0, sg1)
        ssem = (ss0, ss1)

        def body(g, carry):
            grp = c * GROUPS_PER_CORE + g * NS + s
            pltpu.sync_copy(row_hbm.at[pl.ds(grp * 8, 8)], idxr)
            pltpu.sync_copy(col_hbm.at[pl.ds(grp * 8, 8)], idxc)

            def gidx(i):
                return (idxc if i % 2 == 0 else idxr).at[i // 2]

            def sidx(i):
                return (idxr if i % 2 == 0 else idxc).at[i // 2]

            def fire_g(i):
                return pltpu.async_copy(x_hbm.at[gidx(i)], bufs[i % 2],
                                        gsem[i % 2])

            dg = [fire_g(0), fire_g(1)]
            for i in range(16):
                p = i % 2
                dg[p].wait()
                ds_ = pltpu.async_copy(bufs[p], acc_s.at[sidx(i)],
                                       ssem[p], add=True)
                ds_.wait()
                if i + 2 < 16:
                    dg[p] = fire_g(i + 2)
            return carry

        lax.fori_loop(0, GROUPS_PER_TILE, body, 0)
        plsc.subcore_barrier()
        o0 = c * N_NODES + r0

        @pl.when(s < NS - 1)
        def _():
            pltpu.sync_copy(acc_s.at[pl.ds(r0, SLC)],
                            acc_out.at[pl.ds(o0, SLC)])

        @pl.when(s == NS - 1)
        def _():
            pltpu.sync_copy(acc_s.at[pl.ds(r0, SLC_LAST_W)],
                            acc_out.at[pl.ds(o0, SLC_LAST_W)])

    return k(x, row2d, col2d, z128)


def _sc_edge_t(ea_flat, row2d, col2d):
    mesh = plsc.VectorSubcoreMesh(core_axis_name="c", subcore_axis_name="s")

    @functools.partial(
        pl.kernel,
        out_type=jax.ShapeDtypeStruct((NC * NS, TW), jnp.float32),
        mesh=mesh,
        compiler_params=pltpu.CompilerParams(needs_layout_passes=False),
        scratch_types=[
            pltpu.VMEM((TW,), jnp.float32),
            pltpu.VMEM((8, K), jnp.int32),
            pltpu.VMEM((8, K), jnp.int32),
            pltpu.VMEM((8 * K * DE,), jnp.float32),
        ],
    )
    def k(ea_hbm, row_hbm, col_hbm, t_out, t_tile, idxr, idxc, eab):
        c = lax.axis_index("c")
        s = lax.axis_index("s")
        w = c * NS + s
        zero16 = jnp.zeros((16,), jnp.float32)

        def zbody(q, carry):
            t_tile[pl.ds(q * 16, 16)] = zero16
            return carry

        lax.fori_loop(0, TW // 16, zbody, 0)

        def body(g, carry):
            grp = c * GROUPS_PER_CORE + g * NS + s
            pltpu.sync_copy(row_hbm.at[pl.ds(grp * 8, 8)], idxr)
            pltpu.sync_copy(col_hbm.at[pl.ds(grp * 8, 8)], idxc)
            pltpu.sync_copy(ea_hbm.at[pl.ds(grp * (8 * K * DE), 8 * K * DE)],
                            eab)
            for j in range(8):
                for gg in range(8):
                    er = idxr[j, pl.ds(gg * 16, 16)] * DE
                    ec = idxc[j, pl.ds(gg * 16, 16)] * DE
                    ebase = (j * K + gg * 16) * DE
                    for comp in range(DE):
                        vals = eab[pl.ds(ebase + comp * 16, 16)]
                        plsc.addupdate_scatter(t_tile, [er + comp], vals)
                        plsc.addupdate_scatter(t_tile, [ec + comp], vals)
            return carry

        lax.fori_loop(0, GROUPS_PER_TILE, body, 0)
        pltpu.sync_copy(t_tile, t_out.at[w])

    return k(ea_flat, row2d, col2d)


def _tc_tsum(t32):
    def body(a_ref, o_ref):
        o_ref[...] = jnp.sum(a_ref[...], axis=0)

    return pl.pallas_call(
        body,
        grid=(10,),
        in_specs=[pl.BlockSpec((NC * NS, TW // 10), lambda i: (0, i))],
        out_specs=pl.BlockSpec((TW // 10,), lambda i: (i,)),
        out_shape=jax.ShapeDtypeStruct((TW,), jnp.float32),
    )(t32)


BLK = 1000


def _tc_combine(acc2, t, we, wfc):
    def body(a_ref, t_ref, we_ref, wfc_ref, o_ref):
        u = 2.0 * (a_ref[0] + a_ref[1]) + 4.0 * lax.dot_general(
            t_ref[...], we_ref[...], (((1,), (1,)), ((), ())),
            preferred_element_type=jnp.float32)
        o_ref[...] = lax.dot_general(
            u, wfc_ref[...], (((1,), (1,)), ((), ())),
            preferred_element_type=jnp.float32)

    return pl.pallas_call(
        body,
        grid=(N_NODES // BLK,),
        in_specs=[
            pl.BlockSpec((NC, BLK, D), lambda i: (0, i, 0)),
            pl.BlockSpec((BLK, DE), lambda i: (i, 0)),
            pl.BlockSpec((D, DE), lambda i: (0, 0)),
            pl.BlockSpec((D, D), lambda i: (0, 0)),
        ],
        out_specs=pl.BlockSpec((BLK, D), lambda i: (i, 0)),
        out_shape=jax.ShapeDtypeStruct((N_NODES, D), jnp.float32),
    )(acc2, t, we, wfc)


def kernel(x, edge_index, edge_attr, W_fc, W_edge):
    row = edge_index[0].astype(jnp.int32)
    col = edge_index[1].astype(jnp.int32)
    npad = E_PAD - N_EDGES
    pad_idx = N_NODES + (jnp.arange(npad, dtype=jnp.int32) % N_PAD)
    row2d = jnp.concatenate([row, pad_idx]).reshape(E_PAD // K, K)
    col2d = jnp.concatenate([col, pad_idx]).reshape(E_PAD // K, K)
    x_pad = jnp.pad(x, ((0, N_PAD), (0, 0)))
    z128 = jnp.zeros((N_TOT, D), jnp.float32)
    ea_pad = jnp.pad(edge_attr, ((0, npad), (0, 0)))
    ea_flat = ea_pad.reshape(E_PAD // 16, 16, DE).transpose(0, 2, 1).reshape(-1)

    acc_flat = _sc_aggregate(x_pad, row2d, col2d, z128)
    t32 = _sc_edge_t(ea_flat, row2d, col2d)

    tsum = _tc_tsum(t32)
    t = tsum.reshape(TW // DE, DE)[:N_NODES]
    acc2 = acc_flat.reshape(NC, N_NODES, D)
    return _tc_combine(acc2, t, W_edge, W_fc)


# speedup vs baseline: 6.8310x; 1.0130x over previous
import functools

import jax
import jax.numpy as jnp
from jax import lax
from jax.experimental import pallas as pl
from jax.experimental.pallas import tpu as pltpu
from jax.experimental.pallas import tpu_sc as plsc

N_NODES = 10000
N_PAD = 16
N_TOT = 10240
N_EDGES = 320000
E_PAD = 327680
D = 128
DE = 4
K = 128
NC = 2
NS = 16
GROUPS = E_PAD // (8 * K)
GROUPS_PER_CORE = GROUPS // NC
GROUPS_PER_TILE = GROUPS_PER_CORE // NS
SLC = 640
TW = 40960


def _sc_aggregate(x, row2d, col2d):
    mesh = plsc.VectorSubcoreMesh(core_axis_name="c", subcore_axis_name="s")

    @functools.partial(
        pl.kernel,
        out_type=jax.ShapeDtypeStruct((NC * N_TOT, D), jnp.float32),
        mesh=mesh,
        scratch_types=[
            pltpu.VMEM_SHARED((N_TOT, D), jnp.float32),
            pltpu.VMEM((8, K), jnp.int32),
            pltpu.VMEM((8, K), jnp.int32),
            pltpu.VMEM((8, K), jnp.int32),
            pltpu.VMEM((8, K), jnp.int32),
            pltpu.VMEM((K, D), jnp.float32),
            pltpu.VMEM((K, D), jnp.float32),
            pltpu.SemaphoreType.DMA,
            pltpu.SemaphoreType.DMA,
            pltpu.SemaphoreType.DMA,
            pltpu.SemaphoreType.DMA,
            pltpu.SemaphoreType.DMA,
        ],
    )
    def k(x_hbm, row_hbm, col_hbm,
          acc_out,
          acc_s, idxr0, idxc0, idxr1, idxc1, xb0, xb1,
          sg0, sg1, ss0, ss1, si):
        c = lax.axis_index("c")
        s = lax.axis_index("s")
        r0 = s * SLC

        zero16 = jnp.zeros((16,), jnp.float32)

        def zrow(r, carry):
            for cc in range(D // 16):
                xb0[r, pl.ds(cc * 16, 16)] = zero16
            return carry

        lax.fori_loop(0, K, zrow, 0)

        for m in range(SLC // K):
            pltpu.sync_copy(xb0, acc_s.at[pl.ds(r0 + m * K, K)])

        plsc.subcore_barrier()

        bufs = (xb0, xb1)
        gsem = (sg0, sg1)
        ssem = (ss0, ss1)
        idxbufs = ((idxr0, idxc0), (idxr1, idxc1))

        def fetch_idx(g):
            grp = c * GROUPS_PER_CORE + g * NS + s
            ir, ic = idxbufs[g % 2]
            a = pltpu.async_copy(row_hbm.at[pl.ds(grp * 8, 8)], ir, si)
            b = pltpu.async_copy(col_hbm.at[pl.ds(grp * 8, 8)], ic, si)
            return (a, b)

        pending_idx = fetch_idx(0)

        for g in range(GROUPS_PER_TILE):
            pending_idx[0].wait()
            pending_idx[1].wait()
            ir, ic = idxbufs[g % 2]
            if g + 1 < GROUPS_PER_TILE:
                pending_idx = fetch_idx(g + 1)

            def gidx(i):
                return (ic if i % 2 == 0 else ir).at[i // 2]

            def sidx(i):
                return (ir if i % 2 == 0 else ic).at[i // 2]

            def fire_g(i):
                return pltpu.async_copy(x_hbm.at[gidx(i)], bufs[i % 2],
                                        gsem[i % 2])

            dg = [fire_g(0), fire_g(1)]
            for i in range(16):
                p = i % 2
                dg[p].wait()
                ds_ = pltpu.async_copy(bufs[p], acc_s.at[sidx(i)],
                                       ssem[p], add=True)
                ds_.wait()
                if i + 2 < 16:
                    dg[p] = fire_g(i + 2)

        plsc.subcore_barrier()
        o0 = c * N_TOT + r0
        pltpu.sync_copy(acc_s.at[pl.ds(r0, SLC)],
                        acc_out.at[pl.ds(o0, SLC)])

    return k(x, row2d, col2d)


def _sc_edge_t(ea_flat, row2d, col2d):
    mesh = plsc.VectorSubcoreMesh(core_axis_name="c", subcore_axis_name="s")

    @functools.partial(
        pl.kernel,
        out_type=jax.ShapeDtypeStruct((NC * NS, TW), jnp.float32),
        mesh=mesh,
        compiler_params=pltpu.CompilerParams(needs_layout_passes=False),
        scratch_types=[
            pltpu.VMEM((TW,), jnp.float32),
            pltpu.VMEM((8, K), jnp.int32),
            pltpu.VMEM((8, K), jnp.int32),
            pltpu.VMEM((8 * K * DE,), jnp.float32),
        ],
    )
    def k(ea_hbm, row_hbm, col_hbm, t_out, t_tile, idxr, idxc, eab):
        c = lax.axis_index("c")
        s = lax.axis_index("s")
        w = c * NS + s
        zero16 = jnp.zeros((16,), jnp.float32)

        def zbody(q, carry):
            t_tile[pl.ds(q * 16, 16)] = zero16
            return carry

        lax.fori_loop(0, TW // 16, zbody, 0)

        def body(g, carry):
            grp = c * GROUPS_PER_CORE + g * NS + s
            pltpu.sync_copy(row_hbm.at[pl.ds(grp * 8, 8)], idxr)
            pltpu.sync_copy(col_hbm.at[pl.ds(grp * 8, 8)], idxc)
            pltpu.sync_copy(ea_hbm.at[pl.ds(grp * (8 * K * DE), 8 * K * DE)],
                            eab)
            for j in range(8):
                for gg in range(8):
                    er = idxr[j, pl.ds(gg * 16, 16)] * DE
                    ec = idxc[j, pl.ds(gg * 16, 16)] * DE
                    ebase = (j * K + gg * 16) * DE
                    for comp in range(DE):
                        vals = eab[pl.ds(ebase + comp * 16, 16)]
                        plsc.addupdate_scatter(t_tile, [er + comp], vals)
                        plsc.addupdate_scatter(t_tile, [ec + comp], vals)
            return carry

        lax.fori_loop(0, GROUPS_PER_TILE, body, 0)
        pltpu.sync_copy(t_tile, t_out.at[w])

    return k(ea_flat, row2d, col2d)


BLK = 1024


def _tc_tsum(t32):
    def body(a_ref, o_ref):
        o_ref[...] = jnp.sum(a_ref[...], axis=0)

    return pl.pallas_call(
        body,
        grid=(N_TOT // BLK,),
        in_specs=[pl.BlockSpec((NC * NS, BLK * DE), lambda i: (0, i))],
        out_specs=pl.BlockSpec((BLK * DE,), lambda i: (i,)),
        out_shape=jax.ShapeDtypeStruct((TW,), jnp.float32),
    )(t32)


def _tc_combine(acc2, t, we, wfc):
    def body(a_ref, t_ref, we_ref, wfc_ref, o_ref):
        u = 2.0 * (a_ref[0] + a_ref[1]) + 4.0 * lax.dot_general(
            t_ref[...], we_ref[...], (((1,), (1,)), ((), ())),
            preferred_element_type=jnp.float32)
        o_ref[...] = lax.dot_general(
            u, wfc_ref[...], (((1,), (1,)), ((), ())),
            preferred_element_type=jnp.float32)

    return pl.pallas_call(
        body,
        grid=(N_TOT // BLK,),
        in_specs=[
            pl.BlockSpec((NC, BLK, D), lambda i: (0, i, 0)),
            pl.BlockSpec((BLK, DE), lambda i: (i, 0)),
            pl.BlockSpec((D, DE), lambda i: (0, 0)),
            pl.BlockSpec((D, D), lambda i: (0, 0)),
        ],
        out_specs=pl.BlockSpec((BLK, D), lambda i: (i, 0)),
        out_shape=jax.ShapeDtypeStruct((N_TOT, D), jnp.float32),
    )(acc2, t, we, wfc)


def kernel(x, edge_index, edge_attr, W_fc, W_edge):
    row = edge_index[0].astype(jnp.int32)
    col = edge_index[1].astype(jnp.int32)
    npad = E_PAD - N_EDGES
    pad_idx = N_NODES + (jnp.arange(npad, dtype=jnp.int32) % N_PAD)
    row2d = jnp.concatenate([row, pad_idx]).reshape(E_PAD // K, K)
    col2d = jnp.concatenate([col, pad_idx]).reshape(E_PAD // K, K)
    x_pad = jnp.pad(x, ((0, N_TOT - N_NODES), (0, 0)))
    ea_pad = jnp.pad(edge_attr, ((0, npad), (0, 0)))
    ea_flat = ea_pad.reshape(E_PAD // 16, 16, DE).transpose(0, 2, 1).reshape(-1)

    acc_flat = _sc_aggregate(x_pad, row2d, col2d)
    t32 = _sc_edge_t(ea_flat, row2d, col2d)

    t = _tc_tsum(t32).reshape(N_TOT, DE)
    acc2 = acc_flat.reshape(NC, N_TOT, D)
    return _tc_combine(acc2, t, W_edge, W_fc)[:N_NODES]


# speedup vs baseline: 6.9846x; 1.0225x over previous
import functools

import jax
import jax.numpy as jnp
from jax import lax
from jax.experimental import pallas as pl
from jax.experimental.pallas import tpu as pltpu
from jax.experimental.pallas import tpu_sc as plsc

N_NODES = 10000
N_PAD = 16
N_TOT = 10240
N_EDGES = 320000
E_PAD = 327680
D = 128
DE = 4
K = 128
NC = 2
NS = 16
GROUPS = E_PAD // (8 * K)
GROUPS_PER_CORE = GROUPS // NC
GROUPS_PER_TILE = GROUPS_PER_CORE // NS
SLC = 640
TW = 40960


def _sc_aggregate(x, row2d, col2d):
    mesh = plsc.VectorSubcoreMesh(core_axis_name="c", subcore_axis_name="s")

    @functools.partial(
        pl.kernel,
        out_type=jax.ShapeDtypeStruct((NC * N_TOT, D), jnp.float32),
        mesh=mesh,
        scratch_types=[
            pltpu.VMEM_SHARED((N_TOT, D), jnp.float32),
            pltpu.VMEM((8, K), jnp.int32),
            pltpu.VMEM((8, K), jnp.int32),
            pltpu.VMEM((8, K), jnp.int32),
            pltpu.VMEM((8, K), jnp.int32),
            pltpu.VMEM((K, D), jnp.float32),
            pltpu.VMEM((K, D), jnp.float32),
            pltpu.SemaphoreType.DMA,
            pltpu.SemaphoreType.DMA,
            pltpu.SemaphoreType.DMA,
            pltpu.SemaphoreType.DMA,
            pltpu.SemaphoreType.DMA,
        ],
    )
    def k(x_hbm, row_hbm, col_hbm,
          acc_out,
          acc_s, idxr0, idxc0, idxr1, idxc1, xb0, xb1,
          sg0, sg1, ss0, ss1, si):
        c = lax.axis_index("c")
        s = lax.axis_index("s")
        r0 = s * SLC

        zero16 = jnp.zeros((16,), jnp.float32)

        def zrow(r, carry):
            for cc in range(D // 16):
                xb0[r, pl.ds(cc * 16, 16)] = zero16
            return carry

        lax.fori_loop(0, K, zrow, 0)

        for m in range(SLC // K):
            pltpu.sync_copy(xb0, acc_s.at[pl.ds(r0 + m * K, K)])

        plsc.subcore_barrier()

        bufs = (xb0, xb1)
        gsem = (sg0, sg1)
        ssem = (ss0, ss1)
        idxbufs = ((idxr0, idxc0), (idxr1, idxc1))

        def fetch_idx(g):
            grp = c * GROUPS_PER_CORE + g * NS + s
            ir, ic = idxbufs[g % 2]
            a = pltpu.async_copy(row_hbm.at[pl.ds(grp * 8, 8)], ir, si)
            b = pltpu.async_copy(col_hbm.at[pl.ds(grp * 8, 8)], ic, si)
            return (a, b)

        pending_idx = fetch_idx(0)

        for g in range(GROUPS_PER_TILE):
            pending_idx[0].wait()
            pending_idx[1].wait()
            ir, ic = idxbufs[g % 2]
            if g + 1 < GROUPS_PER_TILE:
                pending_idx = fetch_idx(g + 1)

            def gidx(i):
                return (ic if i % 2 == 0 else ir).at[i // 2]

            def sidx(i):
                return (ir if i % 2 == 0 else ic).at[i // 2]

            def fire_g(i):
                return pltpu.async_copy(x_hbm.at[gidx(i)], bufs[i % 2],
                                        gsem[i % 2])

            dg = [fire_g(0), fire_g(1)]
            for i in range(16):
                p = i % 2
                dg[p].wait()
                ds_ = pltpu.async_copy(bufs[p], acc_s.at[sidx(i)],
                                       ssem[p], add=True)
                ds_.wait()
                if i + 2 < 16:
                    dg[p] = fire_g(i + 2)

        plsc.subcore_barrier()
        o0 = c * N_TOT + r0
        pltpu.sync_copy(acc_s.at[pl.ds(r0, SLC)],
                        acc_out.at[pl.ds(o0, SLC)])

    return k(x, row2d, col2d)


def _sc_edge_t(ea_flat, row2d, col2d):
    mesh = plsc.VectorSubcoreMesh(core_axis_name="c", subcore_axis_name="s")

    @functools.partial(
        pl.kernel,
        out_type=jax.ShapeDtypeStruct((NC * NS, TW), jnp.float32),
        mesh=mesh,
        compiler_params=pltpu.CompilerParams(needs_layout_passes=False),
        scratch_types=[
            pltpu.VMEM((TW,), jnp.float32),
            pltpu.VMEM((8, K), jnp.int32),
            pltpu.VMEM((8, K), jnp.int32),
            pltpu.VMEM((8 * K * DE,), jnp.float32),
        ],
    )
    def k(ea_hbm, row_hbm, col_hbm, t_out, t_tile, idxr, idxc, eab):
        c = lax.axis_index("c")
        s = lax.axis_index("s")
        w = c * NS + s
        zero16 = jnp.zeros((16,), jnp.float32)

        def zbody(q, carry):
            t_tile[pl.ds(q * 16, 16)] = zero16
            return carry

        lax.fori_loop(0, TW // 16, zbody, 0)

        def body(g, carry):
            grp = c * GROUPS_PER_CORE + g * NS + s
            pltpu.sync_copy(row_hbm.at[pl.ds(grp * 8, 8)], idxr)
            pltpu.sync_copy(col_hbm.at[pl.ds(grp * 8, 8)], idxc)
            pltpu.sync_copy(ea_hbm.at[pl.ds(grp * (8 * K * DE), 8 * K * DE)],
                            eab)
            for j in range(8):
                for gg in range(8):
                    er = idxr[j, pl.ds(gg * 16, 16)]
                    ec = idxc[j, pl.ds(gg * 16, 16)]
                    ebase = (j * K + gg * 16) * DE
                    for comp in range(DE):
                        vals = eab[pl.ds(ebase + comp * 16, 16)]
                        plsc.addupdate_scatter(
                            t_tile, [er + comp * N_TOT], vals)
                        plsc.addupdate_scatter(
                            t_tile, [ec + comp * N_TOT], vals)
            return carry

        lax.fori_loop(0, GROUPS_PER_TILE, body, 0)
        pltpu.sync_copy(t_tile, t_out.at[w])

    return k(ea_flat, row2d, col2d)


BLK = 1024


def _tc_combine(acc2, t32r, we, wfc):
    def body(a_ref, t_ref, we_ref, wfc_ref, o_ref):
        tsum = jnp.sum(t_ref[...], axis=0)
        u = 2.0 * (a_ref[0] + a_ref[1]) + 4.0 * lax.dot_general(
            tsum, we_ref[...], (((0,), (1,)), ((), ())),
            preferred_element_type=jnp.float32)
        o_ref[...] = lax.dot_general(
            u, wfc_ref[...], (((1,), (1,)), ((), ())),
            preferred_element_type=jnp.float32)

    return pl.pallas_call(
        body,
        grid=(N_TOT // BLK,),
        in_specs=[
            pl.BlockSpec((NC, BLK, D), lambda i: (0, i, 0)),
            pl.BlockSpec((NC * NS, DE, BLK), lambda i: (0, 0, i)),
            pl.BlockSpec((D, DE), lambda i: (0, 0)),
            pl.BlockSpec((D, D), lambda i: (0, 0)),
        ],
        out_specs=pl.BlockSpec((BLK, D), lambda i: (i, 0)),
        out_shape=jax.ShapeDtypeStruct((N_NODES, D), jnp.float32),
    )(acc2, t32r, we, wfc)


def kernel(x, edge_index, edge_attr, W_fc, W_edge):
    row = edge_index[0].astype(jnp.int32)
    col = edge_index[1].astype(jnp.int32)
    npad = E_PAD - N_EDGES
    pad_idx = N_NODES + (jnp.arange(npad, dtype=jnp.int32) % N_PAD)
    row2d = jnp.concatenate([row, pad_idx]).reshape(E_PAD // K, K)
    col2d = jnp.concatenate([col, pad_idx]).reshape(E_PAD // K, K)
    x_pad = jnp.pad(x, ((0, N_TOT - N_NODES), (0, 0)))
    ea_pad = jnp.pad(edge_attr, ((0, npad), (0, 0)))
    ea_flat = ea_pad.reshape(E_PAD // 16, 16, DE).transpose(0, 2, 1).reshape(-1)

    acc_flat = _sc_aggregate(x_pad, row2d, col2d)
    t32 = _sc_edge_t(ea_flat, row2d, col2d)

    t32r = t32.reshape(NC * NS, DE, N_TOT)
    acc2 = acc_flat.reshape(NC, N_TOT, D)
    return _tc_combine(acc2, t32r, W_edge, W_fc)


# speedup vs baseline: 7.2060x; 1.0317x over previous
import functools

import jax
import jax.numpy as jnp
from jax import lax
from jax.experimental import pallas as pl
from jax.experimental.pallas import tpu as pltpu
from jax.experimental.pallas import tpu_sc as plsc

N_NODES = 10000
N_PAD = 16
N_TOT = 10240
N_EDGES = 320000
E_PAD = 327680
D = 128
DE = 4
K = 128
KH = 64
NC = 2
NS = 16
GROUPS = E_PAD // (8 * K)
GROUPS_PER_CORE = GROUPS // NC
GROUPS_PER_TILE = GROUPS_PER_CORE // NS
SLC = 640
TW = 40960


def _sc_aggregate(x, row2d, col2d):
    mesh = plsc.VectorSubcoreMesh(core_axis_name="c", subcore_axis_name="s")

    @functools.partial(
        pl.kernel,
        out_type=jax.ShapeDtypeStruct((NC * N_TOT, D), jnp.float32),
        mesh=mesh,
        scratch_types=[
            pltpu.VMEM_SHARED((N_TOT, D), jnp.float32),
            pltpu.VMEM((8, K), jnp.int32),
            pltpu.VMEM((8, K), jnp.int32),
            pltpu.VMEM((8, K), jnp.int32),
            pltpu.VMEM((8, K), jnp.int32),
            pltpu.VMEM((KH, D), jnp.float32),
            pltpu.VMEM((KH, D), jnp.float32),
            pltpu.VMEM((KH, D), jnp.float32),
            pltpu.VMEM((KH, D), jnp.float32),
            pltpu.SemaphoreType.DMA,
            pltpu.SemaphoreType.DMA,
            pltpu.SemaphoreType.DMA,
            pltpu.SemaphoreType.DMA,
            pltpu.SemaphoreType.DMA,
            pltpu.SemaphoreType.DMA,
            pltpu.SemaphoreType.DMA,
            pltpu.SemaphoreType.DMA,
            pltpu.SemaphoreType.DMA,
        ],
    )
    def k(x_hbm, row_hbm, col_hbm,
          acc_out,
          acc_s, idxr0, idxc0, idxr1, idxc1, xb0, xb1, xb2, xb3,
          sg0, sg1, sg2, sg3, ss0, ss1, ss2, ss3, si):
        c = lax.axis_index("c")
        s = lax.axis_index("s")
        r0 = s * SLC

        zero16 = jnp.zeros((16,), jnp.float32)

        def zrow(r, carry):
            for cc in range(D // 16):
                xb0[r, pl.ds(cc * 16, 16)] = zero16
            return carry

        lax.fori_loop(0, KH, zrow, 0)

        for m in range(SLC // KH):
            pltpu.sync_copy(xb0, acc_s.at[pl.ds(r0 + m * KH, KH)])

        plsc.subcore_barrier()

        bufs = (xb0, xb1, xb2, xb3)
        gsem = (sg0, sg1, sg2, sg3)
        ssem = (ss0, ss1, ss2, ss3)
        idxbufs = ((idxr0, idxc0), (idxr1, idxc1))

        def fetch_idx(g):
            grp = c * GROUPS_PER_CORE + g * NS + s
            ir, ic = idxbufs[g % 2]
            a = pltpu.async_copy(row_hbm.at[pl.ds(grp * 8, 8)], ir, si)
            b = pltpu.async_copy(col_hbm.at[pl.ds(grp * 8, 8)], ic, si)
            return (a, b)

        pending_idx = fetch_idx(0)

        for g in range(GROUPS_PER_TILE):
            pending_idx[0].wait()
            pending_idx[1].wait()
            ir, ic = idxbufs[g % 2]
            if g + 1 < GROUPS_PER_TILE:
                pending_idx = fetch_idx(g + 1)

            def gidx(i):
                return (ic if i % 2 == 0 else ir).at[
                    i // 4, pl.ds(((i // 2) % 2) * KH, KH)]

            def sidx(i):
                return (ir if i % 2 == 0 else ic).at[
                    i // 4, pl.ds(((i // 2) % 2) * KH, KH)]

            def fire_g(i):
                return pltpu.async_copy(x_hbm.at[gidx(i)], bufs[i % 4],
                                        gsem[i % 4])

            dg = [fire_g(0), fire_g(1), fire_g(2), fire_g(3)]
            dsc = [None, None, None, None]
            for i in range(32):
                p = i % 4
                dg[p].wait()
                dsc[p] = pltpu.async_copy(bufs[p], acc_s.at[sidx(i)],
                                          ssem[p], add=True)
                if i + 4 < 32:
                    dsc[p].wait()
                    dg[p] = fire_g(i + 4)
            for p in range(4):
                dsc[p].wait()

        plsc.subcore_barrier()
        o0 = c * N_TOT + r0
        pltpu.sync_copy(acc_s.at[pl.ds(r0, SLC)],
                        acc_out.at[pl.ds(o0, SLC)])

    return k(x, row2d, col2d)


def _sc_edge_t(ea_flat, row2d, col2d):
    mesh = plsc.VectorSubcoreMesh(core_axis_name="c", subcore_axis_name="s")

    @functools.partial(
        pl.kernel,
        out_type=jax.ShapeDtypeStruct((NC * NS, TW), jnp.float32),
        mesh=mesh,
        compiler_params=pltpu.CompilerParams(needs_layout_passes=False),
        scratch_types=[
            pltpu.VMEM((TW,), jnp.float32),
            pltpu.VMEM((8, K), jnp.int32),
            pltpu.VMEM((8, K), jnp.int32),
            pltpu.VMEM((8 * K * DE,), jnp.float32),
        ],
    )
    def k(ea_hbm, row_hbm, col_hbm, t_out, t_tile, idxr, idxc, eab):
        c = lax.axis_index("c")
        s = lax.axis_index("s")
        w = c * NS + s
        zero16 = jnp.zeros((16,), jnp.float32)

        def zbody(q, carry):
            t_tile[pl.ds(q * 16, 16)] = zero16
            return carry

        lax.fori_loop(0, TW // 16, zbody, 0)

        def body(g, carry):
            grp = c * GROUPS_PER_CORE + g * NS + s
            pltpu.sync_copy(row_hbm.at[pl.ds(grp * 8, 8)], idxr)
            pltpu.sync_copy(col_hbm.at[pl.ds(grp * 8, 8)], idxc)
            pltpu.sync_copy(ea_hbm.at[pl.ds(grp * (8 * K * DE), 8 * K * DE)],
                            eab)
            for j in range(8):
                for gg in range(8):
                    er = idxr[j, pl.ds(gg * 16, 16)]
                    ec = idxc[j, pl.ds(gg * 16, 16)]
                    ebase = (j * K + gg * 16) * DE
                    for comp in range(DE):
                        vals = eab[pl.ds(ebase + comp * 16, 16)]
                        plsc.addupdate_scatter(
                            t_tile, [er + comp * N_TOT], vals)
                        plsc.addupdate_scatter(
                            t_tile, [ec + comp * N_TOT], vals)
            return carry

        lax.fori_loop(0, GROUPS_PER_TILE, body, 0)
        pltpu.sync_copy(t_tile, t_out.at[w])

    return k(ea_flat, row2d, col2d)


BLK = 1024


def _tc_combine(acc2, t32r, we, wfc):
    def body(a_ref, t_ref, we_ref, wfc_ref, o_ref):
        tsum = jnp.sum(t_ref[...], axis=0)
        u = 2.0 * (a_ref[0] + a_ref[1]) + 4.0 * lax.dot_general(
            tsum, we_ref[...], (((0,), (1,)), ((), ())),
            preferred_element_type=jnp.float32)
        o_ref[...] = lax.dot_general(
            u, wfc_ref[...], (((1,), (1,)), ((), ())),
            preferred_element_type=jnp.float32)

    return pl.pallas_call(
        body,
        grid=(N_TOT // BLK,),
        in_specs=[
            pl.BlockSpec((NC, BLK, D), lambda i: (0, i, 0)),
            pl.BlockSpec((NC * NS, DE, BLK), lambda i: (0, 0, i)),
            pl.BlockSpec((D, DE), lambda i: (0, 0)),
            pl.BlockSpec((D, D), lambda i: (0, 0)),
        ],
        out_specs=pl.BlockSpec((BLK, D), lambda i: (i, 0)),
        out_shape=jax.ShapeDtypeStruct((N_NODES, D), jnp.float32),
    )(acc2, t32r, we, wfc)


def kernel(x, edge_index, edge_attr, W_fc, W_edge):
    row = edge_index[0].astype(jnp.int32)
    col = edge_index[1].astype(jnp.int32)
    npad = E_PAD - N_EDGES
    pad_idx = N_NODES + (jnp.arange(npad, dtype=jnp.int32) % N_PAD)
    row2d = jnp.concatenate([row, pad_idx]).reshape(E_PAD // K, K)
    col2d = jnp.concatenate([col, pad_idx]).reshape(E_PAD // K, K)
    x_pad = jnp.pad(x, ((0, N_TOT - N_NODES), (0, 0)))
    ea_pad = jnp.pad(edge_attr, ((0, npad), (0, 0)))
    ea_flat = ea_pad.reshape(E_PAD // 16, 16, DE).transpose(0, 2, 1).reshape(-1)

    acc_flat = _sc_aggregate(x_pad, row2d, col2d)
    t32 = _sc_edge_t(ea_flat, row2d, col2d)

    t32r = t32.reshape(NC * NS, DE, N_TOT)
    acc2 = acc_flat.reshape(NC, N_TOT, D)
    return _tc_combine(acc2, t32r, W_edge, W_fc)
